# baseline (speedup 1.0000x reference)
import functools

import numpy as np
import jax
import jax.numpy as jnp
from jax import lax
from jax.experimental import pallas as pl
from jax.experimental.pallas import tpu as pltpu
from jax.experimental.pallas import tpu_sc as plsc

_M32 = 0xFFFFFFFF


def _threefry2x32_scalar(k1, k2, x0, x1):
    rot1 = (13, 15, 26, 6)
    rot2 = (17, 29, 16, 24)
    ks = (k1, k2, k1 ^ k2 ^ 0x1BD11BDA)

    def rounds(x0, x1, rots):
        for r in rots:
            x0 = (x0 + x1) & _M32
            x1 = ((x1 << r) | (x1 >> (32 - r))) & _M32
            x1 = x0 ^ x1
        return x0, x1

    x0 = (x0 + ks[0]) & _M32
    x1 = (x1 + ks[1]) & _M32
    x0, x1 = rounds(x0, x1, rot1)
    x0 = (x0 + ks[1]) & _M32
    x1 = (x1 + ks[2] + 1) & _M32
    x0, x1 = rounds(x0, x1, rot2)
    x0 = (x0 + ks[2]) & _M32
    x1 = (x1 + ks[0] + 2) & _M32
    x0, x1 = rounds(x0, x1, rot1)
    x0 = (x0 + ks[0]) & _M32
    x1 = (x1 + ks[1] + 3) & _M32
    x0, x1 = rounds(x0, x1, rot2)
    x0 = (x0 + ks[1]) & _M32
    x1 = (x1 + ks[2] + 4) & _M32
    x0, x1 = rounds(x0, x1, rot1)
    x0 = (x0 + ks[2]) & _M32
    x1 = (x1 + ks[0] + 5) & _M32
    return x0, x1


_K1, _K2 = _threefry2x32_scalar(0, 0, 0, 1)
_KS2 = _K1 ^ _K2 ^ 0x1BD11BDA

_ROT1 = (13, 15, 26, 6)
_ROT2 = (17, 29, 16, 24)

_TINY = np.float32(np.finfo(np.float32).tiny)
_IMAX = np.int32(0x7FFFFFFF)


def _u32(v):
    return np.uint32(v & _M32)


def _rounds(x0, x1, rots):
    for r in rots:
        x0 = x0 + x1
        x1 = lax.shift_left(x1, np.uint32(r)) | lax.shift_right_logical(
            x1, np.uint32(32 - r))
        x1 = x0 ^ x1
    return x0, x1


def _threefry_fb(flat_idx_u32):
    x1 = flat_idx_u32 + _u32(_K2)
    x0 = jnp.zeros_like(x1) + _u32(_K1)
    x0, x1 = _rounds(x0, x1, _ROT1)
    x0 = x0 + _u32(_K2)
    x1 = x1 + _u32(_KS2 + 1)
    x0, x1 = _rounds(x0, x1, _ROT2)
    x0 = x0 + _u32(_KS2)
    x1 = x1 + _u32(_K1 + 2)
    x0, x1 = _rounds(x0, x1, _ROT1)
    x0 = x0 + _u32(_K1)
    x1 = x1 + _u32(_K2 + 3)
    x0, x1 = _rounds(x0, x1, _ROT2)
    x0 = x0 + _u32(_K2)
    x1 = x1 + _u32(_KS2 + 4)
    x0, x1 = _rounds(x0, x1, _ROT1)
    x0 = x0 + _u32(_KS2)
    x1 = x1 + _u32(_K1 + 5)
    bits = x0 ^ x1
    return lax.shift_right_logical(bits, np.uint32(9)) | np.uint32(0x3F800000)


def _fb_val(fb, blk):
    f = lax.bitcast_convert_type(fb, jnp.float32) - np.float32(1.0)
    u = jnp.maximum(f, _TINY)
    return blk - jnp.log(-jnp.log(u))


_CW = 128


def _tc_main_block(logits_ref, accv_ref, accc_ref, *, n, c_lo, w):
    i = pl.program_id(0)
    nb = pl.num_programs(0)
    b = logits_ref.shape[0]

    @pl.when(i == 0)
    def _():
        accv_ref[...] = jnp.full((b, _CW), -jnp.inf, jnp.float32)
        accc_ref[...] = jnp.zeros((b, _CW), jnp.int32)

    def run(masked):
        acc_v = accv_ref[...]
        acc_c = accc_ref[...]
        col0 = lax.broadcasted_iota(jnp.int32, (b, _CW), 1) + (c_lo + i * w)
        row = lax.broadcasted_iota(jnp.uint32, (b, _CW), 0)
        flat0 = row * np.uint32(n) + lax.convert_element_type(col0, jnp.uint32)
        for j in range(w // _CW):
            blk = logits_ref[:, j * _CW:(j + 1) * _CW]
            val = _fb_val(_threefry_fb(flat0 + np.uint32(j * _CW)), blk)
            colj = col0 + np.int32(j * _CW)
            if masked:
                val = jnp.where(colj < n, val, -jnp.inf)
            upd = val > acc_v
            acc_v = jnp.where(upd, val, acc_v)
            acc_c = jnp.where(upd, colj, acc_c)
        accv_ref[...] = acc_v
        accc_ref[...] = acc_c

    if (n - c_lo) % w == 0:
        run(False)
    else:
        @pl.when(i < nb - 1)
        def _():
            run(False)

        @pl.when(i == nb - 1)
        def _():
            run(True)


def _sc_noise(b, n, n_sc):
    nblk = n_sc // _CW
    blocks_per_sub = nblk // 32

    def body(o_ref, tile0_ref, tile1_ref, sem0, sem1):
        core = lax.axis_index("c")
        sub = lax.axis_index("s")
        sid = core * 16 + sub
        iota16 = lax.iota(jnp.int32, 16)
        tiles = (tile0_ref, tile1_ref)
        sems = (sem0, sem1)
        cps = [None, None]

        for t in range(blocks_per_sub):
            tile_ref = tiles[t % 2]
            if cps[t % 2] is not None:
                cps[t % 2].wait()
            blk = sid * blocks_per_sub + t
            c0 = blk * _CW

            @pl.loop(0, b)
            def _(r, c0=c0, tile_ref=tile_ref):
                base = r * n + c0
                for u in range(8):
                    idx = base + u * 16 + iota16
                    fb = _threefry_fb(plsc.bitcast(idx, jnp.uint32))
                    tile_ref[r, pl.ds(u * 16, 16)] = fb

            cps[t % 2] = pltpu.async_copy(tile_ref, o_ref.at[blk], sems[t % 2])
        for cp in cps:
            if cp is not None:
                cp.wait()

    mesh = plsc.VectorSubcoreMesh(core_axis_name="c", subcore_axis_name="s")
    return pl.kernel(
        body,
        out_type=jax.ShapeDtypeStruct((nblk, b, _CW), jnp.uint32),
        mesh=mesh,
        scratch_types=[
            pltpu.VMEM((b, _CW), jnp.uint32),
            pltpu.VMEM((b, _CW), jnp.uint32),
            pltpu.SemaphoreType.DMA,
            pltpu.SemaphoreType.DMA,
        ],
    )()


def _tc_tail_block(logits_ref, fb_ref, accv_in_ref, accc_in_ref, out_ref,
                   accv_s, accc_s, *, w):
    i = pl.program_id(0)
    nb = pl.num_programs(0)
    b = logits_ref.shape[0]

    @pl.when(i == 0)
    def _():
        accv_s[...] = accv_in_ref[...]
        accc_s[...] = accc_in_ref[...]

    acc_v = accv_s[...]
    acc_c = accc_s[...]

    col0 = lax.broadcasted_iota(jnp.int32, (b, _CW), 1) + i * w

    for j in range(w // _CW):
        blk = logits_ref[:, j * _CW:(j + 1) * _CW]
        val = _fb_val(fb_ref[j], blk)
        colj = col0 + np.int32(j * _CW)
        upd = (val > acc_v) | ((val == acc_v) & (colj < acc_c))
        acc_v = jnp.where(upd, val, acc_v)
        acc_c = jnp.where(upd, colj, acc_c)

    accv_s[...] = acc_v
    accc_s[...] = acc_c

    @pl.when(i == nb - 1)
    def _():
        m = jnp.max(acc_v, axis=1, keepdims=True)
        idx = jnp.min(jnp.where(acc_v == m, acc_c, _IMAX),
                      axis=1, keepdims=True)
        out_ref[...] = idx


def _tc_only_block(logits_ref, out_ref, accv_ref, accc_ref, *, n, w):
    i = pl.program_id(0)
    nb = pl.num_programs(0)
    b = logits_ref.shape[0]
    _tc_main_block(logits_ref, accv_ref, accc_ref, n=n, c_lo=0, w=w)

    @pl.when(i == nb - 1)
    def _():
        acc_v = accv_ref[...]
        m = jnp.max(acc_v, axis=1, keepdims=True)
        idx = jnp.min(jnp.where(acc_v == m, accc_ref[...], _IMAX),
                      axis=1, keepdims=True)
        out_ref[...] = idx


def _kernel_tc_only(logits):
    b, n = logits.shape
    w = min(8192, -(-n // _CW) * _CW)
    nb = -(-n // w)
    out = pl.pallas_call(
        functools.partial(_tc_only_block, n=n, w=w),
        grid=(nb,),
        in_specs=[pl.BlockSpec((b, w), lambda i: (0, i))],
        out_specs=pl.BlockSpec((b, 1), lambda i: (0, 0)),
        out_shape=jax.ShapeDtypeStruct((b, 1), jnp.int32),
        scratch_shapes=[
            pltpu.VMEM((b, _CW), jnp.float32),
            pltpu.VMEM((b, _CW), jnp.int32),
        ],
    )(logits)
    return out[:, 0]


@jax.jit
def kernel(logits):
    b, n = logits.shape
    if (b, n) != (64, 100000):
        return _kernel_tc_only(logits)

    n_sc = 28672
    w_a = 2048
    nb_a = -(-(n - n_sc) // w_a)
    w_b = 4096
    nb_b = n_sc // w_b

    fb = _sc_noise(b, n, n_sc)

    accv, accc = pl.pallas_call(
        functools.partial(_tc_main_block, n=n, c_lo=n_sc, w=w_a),
        grid=(nb_a,),
        in_specs=[pl.BlockSpec((b, w_a), lambda i: (0, i + n_sc // w_a))],
        out_specs=[pl.BlockSpec((b, _CW), lambda i: (0, 0)),
                   pl.BlockSpec((b, _CW), lambda i: (0, 0))],
        out_shape=[jax.ShapeDtypeStruct((b, _CW), jnp.float32),
                   jax.ShapeDtypeStruct((b, _CW), jnp.int32)],
    )(logits)

    out = pl.pallas_call(
        functools.partial(_tc_tail_block, w=w_b),
        grid=(nb_b,),
        in_specs=[
            pl.BlockSpec((b, w_b), lambda i: (0, i)),
            pl.BlockSpec((w_b // _CW, b, _CW), lambda i: (i, 0, 0)),
            pl.BlockSpec((b, _CW), lambda i: (0, 0)),
            pl.BlockSpec((b, _CW), lambda i: (0, 0)),
        ],
        out_specs=pl.BlockSpec((b, 1), lambda i: (0, 0)),
        out_shape=jax.ShapeDtypeStruct((b, 1), jnp.int32),
        scratch_shapes=[
            pltpu.VMEM((b, _CW), jnp.float32),
            pltpu.VMEM((b, _CW), jnp.int32),
        ],
    )(logits, fb, accv, accc)
    return out[:, 0]


# speedup vs baseline: 1.1149x; 1.1149x over previous
import functools

import numpy as np
import jax
import jax.numpy as jnp
from jax import lax
from jax.experimental import pallas as pl
from jax.experimental.pallas import tpu as pltpu
from jax.experimental.pallas import tpu_sc as plsc

_M32 = 0xFFFFFFFF


def _threefry2x32_scalar(k1, k2, x0, x1):
    rot1 = (13, 15, 26, 6)
    rot2 = (17, 29, 16, 24)
    ks = (k1, k2, k1 ^ k2 ^ 0x1BD11BDA)

    def rounds(x0, x1, rots):
        for r in rots:
            x0 = (x0 + x1) & _M32
            x1 = ((x1 << r) | (x1 >> (32 - r))) & _M32
            x1 = x0 ^ x1
        return x0, x1

    x0 = (x0 + ks[0]) & _M32
    x1 = (x1 + ks[1]) & _M32
    x0, x1 = rounds(x0, x1, rot1)
    x0 = (x0 + ks[1]) & _M32
    x1 = (x1 + ks[2] + 1) & _M32
    x0, x1 = rounds(x0, x1, rot2)
    x0 = (x0 + ks[2]) & _M32
    x1 = (x1 + ks[0] + 2) & _M32
    x0, x1 = rounds(x0, x1, rot1)
    x0 = (x0 + ks[0]) & _M32
    x1 = (x1 + ks[1] + 3) & _M32
    x0, x1 = rounds(x0, x1, rot2)
    x0 = (x0 + ks[1]) & _M32
    x1 = (x1 + ks[2] + 4) & _M32
    x0, x1 = rounds(x0, x1, rot1)
    x0 = (x0 + ks[2]) & _M32
    x1 = (x1 + ks[0] + 5) & _M32
    return x0, x1


_K1, _K2 = _threefry2x32_scalar(0, 0, 0, 1)
_KS2 = _K1 ^ _K2 ^ 0x1BD11BDA

_ROT1 = (13, 15, 26, 6)
_ROT2 = (17, 29, 16, 24)

_TINY = np.float32(np.finfo(np.float32).tiny)
_IMAX = np.int32(0x7FFFFFFF)


def _u32(v):
    return np.uint32(v & _M32)


def _rounds(x0, x1, rots):
    for r in rots:
        x0 = x0 + x1
        x1 = lax.shift_left(x1, np.uint32(r)) | lax.shift_right_logical(
            x1, np.uint32(32 - r))
        x1 = x0 ^ x1
    return x0, x1


def _threefry_fb(flat_idx_u32):
    x1 = flat_idx_u32 + _u32(_K2)
    x0 = jnp.zeros_like(x1) + _u32(_K1)
    x0, x1 = _rounds(x0, x1, _ROT1)
    x0 = x0 + _u32(_K2)
    x1 = x1 + _u32(_KS2 + 1)
    x0, x1 = _rounds(x0, x1, _ROT2)
    x0 = x0 + _u32(_KS2)
    x1 = x1 + _u32(_K1 + 2)
    x0, x1 = _rounds(x0, x1, _ROT1)
    x0 = x0 + _u32(_K1)
    x1 = x1 + _u32(_K2 + 3)
    x0, x1 = _rounds(x0, x1, _ROT2)
    x0 = x0 + _u32(_K2)
    x1 = x1 + _u32(_KS2 + 4)
    x0, x1 = _rounds(x0, x1, _ROT1)
    x0 = x0 + _u32(_KS2)
    x1 = x1 + _u32(_K1 + 5)
    bits = x0 ^ x1
    return lax.shift_right_logical(bits, np.uint32(9)) | np.uint32(0x3F800000)


def _fb_val(fb, blk):
    f = lax.bitcast_convert_type(fb, jnp.float32) - np.float32(1.0)
    u = jnp.maximum(f, _TINY)
    return blk - jnp.log(-jnp.log(u))


_CW = 128


def _tc_main_block(logits_ref, accv_ref, accc_ref, *, n, c_lo, w):
    i = pl.program_id(0)
    nb = pl.num_programs(0)
    b = logits_ref.shape[0]

    @pl.when(i == 0)
    def _():
        accv_ref[...] = jnp.full((b, _CW), -jnp.inf, jnp.float32)
        accc_ref[...] = jnp.zeros((b, _CW), jnp.int32)

    def run(masked):
        acc_v = accv_ref[...]
        acc_c = accc_ref[...]
        col0 = lax.broadcasted_iota(jnp.int32, (b, _CW), 1) + (c_lo + i * w)
        row = lax.broadcasted_iota(jnp.uint32, (b, _CW), 0)
        flat0 = row * np.uint32(n) + lax.convert_element_type(col0, jnp.uint32)
        for j in range(w // _CW):
            blk = logits_ref[:, j * _CW:(j + 1) * _CW]
            val = _fb_val(_threefry_fb(flat0 + np.uint32(j * _CW)), blk)
            colj = col0 + np.int32(j * _CW)
            if masked:
                val = jnp.where(colj < n, val, -jnp.inf)
            upd = val > acc_v
            acc_v = jnp.where(upd, val, acc_v)
            acc_c = jnp.where(upd, colj, acc_c)
        accv_ref[...] = acc_v
        accc_ref[...] = acc_c

    if (n - c_lo) % w == 0:
        run(False)
    else:
        @pl.when(i < nb - 1)
        def _():
            run(False)

        @pl.when(i == nb - 1)
        def _():
            run(True)


def _sc_noise(b, n, n_sc):
    nblk = n_sc // _CW
    blocks_per_sub = nblk // 32

    def body(o_ref, tile_ref, sem):
        core = lax.axis_index("c")
        sub = lax.axis_index("s")
        sid = core * 16 + sub
        iota16 = lax.iota(jnp.int32, 16)

        @pl.loop(0, blocks_per_sub)
        def _(t):
            blk = sid * blocks_per_sub + t
            c0 = blk * _CW

            @pl.loop(0, b)
            def _(r):
                base = r * n + c0

                @pl.loop(0, _CW, step=64)
                def _(cc):
                    for u in range(4):
                        idx = base + (cc + u * 16) + iota16
                        fb = _threefry_fb(plsc.bitcast(idx, jnp.uint32))
                        tile_ref[r, pl.ds(cc + u * 16, 16)] = fb

            cp = pltpu.async_copy(tile_ref, o_ref.at[blk], sem)
            cp.wait()

    mesh = plsc.VectorSubcoreMesh(core_axis_name="c", subcore_axis_name="s")
    return pl.kernel(
        body,
        out_type=jax.ShapeDtypeStruct((nblk, b, _CW), jnp.uint32),
        mesh=mesh,
        scratch_types=[
            pltpu.VMEM((b, _CW), jnp.uint32),
            pltpu.SemaphoreType.DMA,
        ],
    )()


def _tc_tail_block(logits_ref, fb_ref, accv_in_ref, accc_in_ref, out_ref,
                   accv_s, accc_s, *, w):
    i = pl.program_id(0)
    nb = pl.num_programs(0)
    b = logits_ref.shape[0]

    @pl.when(i == 0)
    def _():
        accv_s[...] = accv_in_ref[...]
        accc_s[...] = accc_in_ref[...]

    acc_v = accv_s[...]
    acc_c = accc_s[...]

    col0 = lax.broadcasted_iota(jnp.int32, (b, _CW), 1) + i * w

    for j in range(w // _CW):
        blk = logits_ref[:, j * _CW:(j + 1) * _CW]
        val = _fb_val(fb_ref[j], blk)
        colj = col0 + np.int32(j * _CW)
        upd = (val > acc_v) | ((val == acc_v) & (colj < acc_c))
        acc_v = jnp.where(upd, val, acc_v)
        acc_c = jnp.where(upd, colj, acc_c)

    accv_s[...] = acc_v
    accc_s[...] = acc_c

    @pl.when(i == nb - 1)
    def _():
        m = jnp.max(acc_v, axis=1, keepdims=True)
        idx = jnp.min(jnp.where(acc_v == m, acc_c, _IMAX),
                      axis=1, keepdims=True)
        out_ref[...] = idx


def _tc_only_block(logits_ref, out_ref, accv_ref, accc_ref, *, n, w):
    i = pl.program_id(0)
    nb = pl.num_programs(0)
    b = logits_ref.shape[0]
    _tc_main_block(logits_ref, accv_ref, accc_ref, n=n, c_lo=0, w=w)

    @pl.when(i == nb - 1)
    def _():
        acc_v = accv_ref[...]
        m = jnp.max(acc_v, axis=1, keepdims=True)
        idx = jnp.min(jnp.where(acc_v == m, accc_ref[...], _IMAX),
                      axis=1, keepdims=True)
        out_ref[...] = idx


def _kernel_tc_only(logits):
    b, n = logits.shape
    w = min(8192, -(-n // _CW) * _CW)
    nb = -(-n // w)
    out = pl.pallas_call(
        functools.partial(_tc_only_block, n=n, w=w),
        grid=(nb,),
        in_specs=[pl.BlockSpec((b, w), lambda i: (0, i))],
        out_specs=pl.BlockSpec((b, 1), lambda i: (0, 0)),
        out_shape=jax.ShapeDtypeStruct((b, 1), jnp.int32),
        scratch_shapes=[
            pltpu.VMEM((b, _CW), jnp.float32),
            pltpu.VMEM((b, _CW), jnp.int32),
        ],
    )(logits)
    return out[:, 0]


@jax.jit
def kernel(logits):
    b, n = logits.shape
    if (b, n) != (64, 100000):
        return _kernel_tc_only(logits)

    n_sc = 24576
    w_a = 2048
    nb_a = -(-(n - n_sc) // w_a)
    w_b = 4096
    nb_b = n_sc // w_b

    fb = _sc_noise(b, n, n_sc)

    accv, accc = pl.pallas_call(
        functools.partial(_tc_main_block, n=n, c_lo=n_sc, w=w_a),
        grid=(nb_a,),
        in_specs=[pl.BlockSpec((b, w_a), lambda i: (0, i + n_sc // w_a))],
        out_specs=[pl.BlockSpec((b, _CW), lambda i: (0, 0)),
                   pl.BlockSpec((b, _CW), lambda i: (0, 0))],
        out_shape=[jax.ShapeDtypeStruct((b, _CW), jnp.float32),
                   jax.ShapeDtypeStruct((b, _CW), jnp.int32)],
    )(logits)

    out = pl.pallas_call(
        functools.partial(_tc_tail_block, w=w_b),
        grid=(nb_b,),
        in_specs=[
            pl.BlockSpec((b, w_b), lambda i: (0, i)),
            pl.BlockSpec((w_b // _CW, b, _CW), lambda i: (i, 0, 0)),
            pl.BlockSpec((b, _CW), lambda i: (0, 0)),
            pl.BlockSpec((b, _CW), lambda i: (0, 0)),
        ],
        out_specs=pl.BlockSpec((b, 1), lambda i: (0, 0)),
        out_shape=jax.ShapeDtypeStruct((b, 1), jnp.int32),
        scratch_shapes=[
            pltpu.VMEM((b, _CW), jnp.float32),
            pltpu.VMEM((b, _CW), jnp.int32),
        ],
    )(logits, fb, accv, accc)
    return out[:, 0]


# speedup vs baseline: 1.1370x; 1.0198x over previous
import functools

import numpy as np
import jax
import jax.numpy as jnp
from jax import lax
from jax.experimental import pallas as pl
from jax.experimental.pallas import tpu as pltpu
from jax.experimental.pallas import tpu_sc as plsc

_M32 = 0xFFFFFFFF


def _threefry2x32_scalar(k1, k2, x0, x1):
    rot1 = (13, 15, 26, 6)
    rot2 = (17, 29, 16, 24)
    ks = (k1, k2, k1 ^ k2 ^ 0x1BD11BDA)

    def rounds(x0, x1, rots):
        for r in rots:
            x0 = (x0 + x1) & _M32
            x1 = ((x1 << r) | (x1 >> (32 - r))) & _M32
            x1 = x0 ^ x1
        return x0, x1

    x0 = (x0 + ks[0]) & _M32
    x1 = (x1 + ks[1]) & _M32
    x0, x1 = rounds(x0, x1, rot1)
    x0 = (x0 + ks[1]) & _M32
    x1 = (x1 + ks[2] + 1) & _M32
    x0, x1 = rounds(x0, x1, rot2)
    x0 = (x0 + ks[2]) & _M32
    x1 = (x1 + ks[0] + 2) & _M32
    x0, x1 = rounds(x0, x1, rot1)
    x0 = (x0 + ks[0]) & _M32
    x1 = (x1 + ks[1] + 3) & _M32
    x0, x1 = rounds(x0, x1, rot2)
    x0 = (x0 + ks[1]) & _M32
    x1 = (x1 + ks[2] + 4) & _M32
    x0, x1 = rounds(x0, x1, rot1)
    x0 = (x0 + ks[2]) & _M32
    x1 = (x1 + ks[0] + 5) & _M32
    return x0, x1


_K1, _K2 = _threefry2x32_scalar(0, 0, 0, 1)
_KS2 = _K1 ^ _K2 ^ 0x1BD11BDA

_ROT1 = (13, 15, 26, 6)
_ROT2 = (17, 29, 16, 24)

_TINY = np.float32(np.finfo(np.float32).tiny)
_IMAX = np.int32(0x7FFFFFFF)


def _u32(v):
    return np.uint32(v & _M32)


def _rounds(x0, x1, rots):
    for r in rots:
        x0 = x0 + x1
        x1 = lax.shift_left(x1, np.uint32(r)) | lax.shift_right_logical(
            x1, np.uint32(32 - r))
        x1 = x0 ^ x1
    return x0, x1


def _threefry_fb(flat_idx_u32):
    x1 = flat_idx_u32 + _u32(_K2)
    x0 = jnp.zeros_like(x1) + _u32(_K1)
    x0, x1 = _rounds(x0, x1, _ROT1)
    x0 = x0 + _u32(_K2)
    x1 = x1 + _u32(_KS2 + 1)
    x0, x1 = _rounds(x0, x1, _ROT2)
    x0 = x0 + _u32(_KS2)
    x1 = x1 + _u32(_K1 + 2)
    x0, x1 = _rounds(x0, x1, _ROT1)
    x0 = x0 + _u32(_K1)
    x1 = x1 + _u32(_K2 + 3)
    x0, x1 = _rounds(x0, x1, _ROT2)
    x0 = x0 + _u32(_K2)
    x1 = x1 + _u32(_KS2 + 4)
    x0, x1 = _rounds(x0, x1, _ROT1)
    x0 = x0 + _u32(_KS2)
    x1 = x1 + _u32(_K1 + 5)
    bits = x0 ^ x1
    return lax.shift_right_logical(bits, np.uint32(9)) | np.uint32(0x3F800000)


def _fb_val(fb, blk):
    f = lax.bitcast_convert_type(fb, jnp.float32) - np.float32(1.0)
    u = jnp.maximum(f, _TINY)
    return blk - jnp.log(-jnp.log(u))


_CW = 128


def _tc_main_block(logits_ref, accv_ref, accc_ref, *, n, c_lo, w):
    i = pl.program_id(0)
    nb = pl.num_programs(0)
    b = logits_ref.shape[0]

    @pl.when(i == 0)
    def _():
        accv_ref[...] = jnp.full((b, _CW), -jnp.inf, jnp.float32)
        accc_ref[...] = jnp.zeros((b, _CW), jnp.int32)

    def run(masked):
        acc_v = accv_ref[...]
        acc_c = accc_ref[...]
        col0 = lax.broadcasted_iota(jnp.int32, (b, _CW), 1) + (c_lo + i * w)
        row = lax.broadcasted_iota(jnp.uint32, (b, _CW), 0)
        flat0 = row * np.uint32(n) + lax.convert_element_type(col0, jnp.uint32)
        for j in range(w // _CW):
            blk = logits_ref[:, j * _CW:(j + 1) * _CW]
            val = _fb_val(_threefry_fb(flat0 + np.uint32(j * _CW)), blk)
            colj = col0 + np.int32(j * _CW)
            if masked:
                val = jnp.where(colj < n, val, -jnp.inf)
            upd = val > acc_v
            acc_v = jnp.where(upd, val, acc_v)
            acc_c = jnp.where(upd, colj, acc_c)
        accv_ref[...] = acc_v
        accc_ref[...] = acc_c

    if (n - c_lo) % w == 0:
        run(False)
    else:
        @pl.when(i < nb - 1)
        def _():
            run(False)

        @pl.when(i == nb - 1)
        def _():
            run(True)


def _sc_noise(b, n, n_sc):
    nblk = n_sc // _CW
    blocks_per_sub = nblk // 32

    def body(o_ref, tile_ref, sem):
        core = lax.axis_index("c")
        sub = lax.axis_index("s")
        sid = core * 16 + sub
        iota16 = lax.iota(jnp.int32, 16)

        @pl.loop(0, blocks_per_sub)
        def _(t):
            blk = sid * blocks_per_sub + t
            c0 = blk * _CW

            @pl.loop(0, b)
            def _(r):
                base = r * n + c0

                @pl.loop(0, _CW, step=64)
                def _(cc):
                    for u in range(4):
                        idx = base + (cc + u * 16) + iota16
                        fb = _threefry_fb(plsc.bitcast(idx, jnp.uint32))
                        tile_ref[r, pl.ds(cc + u * 16, 16)] = fb

            cp = pltpu.async_copy(tile_ref, o_ref.at[blk], sem)
            cp.wait()

    mesh = plsc.VectorSubcoreMesh(core_axis_name="c", subcore_axis_name="s")
    return pl.kernel(
        body,
        out_type=jax.ShapeDtypeStruct((nblk, b, _CW), jnp.uint32),
        mesh=mesh,
        scratch_types=[
            pltpu.VMEM((b, _CW), jnp.uint32),
            pltpu.SemaphoreType.DMA,
        ],
    )()


def _tc_tail_block(logits_ref, fb_ref, accv_in_ref, accc_in_ref, out_ref,
                   accv_s, accc_s, *, w):
    i = pl.program_id(0)
    nb = pl.num_programs(0)
    b = logits_ref.shape[0]

    @pl.when(i == 0)
    def _():
        accv_s[...] = accv_in_ref[...]
        accc_s[...] = accc_in_ref[...]

    acc_v = accv_s[...]
    acc_c = accc_s[...]

    col0 = lax.broadcasted_iota(jnp.int32, (b, _CW), 1) + i * w

    for j in range(w // _CW):
        blk = logits_ref[:, j * _CW:(j + 1) * _CW]
        val = _fb_val(fb_ref[j], blk)
        colj = col0 + np.int32(j * _CW)
        upd = (val > acc_v) | ((val == acc_v) & (colj < acc_c))
        acc_v = jnp.where(upd, val, acc_v)
        acc_c = jnp.where(upd, colj, acc_c)

    accv_s[...] = acc_v
    accc_s[...] = acc_c

    @pl.when(i == nb - 1)
    def _():
        m = jnp.max(acc_v, axis=1, keepdims=True)
        idx = jnp.min(jnp.where(acc_v == m, acc_c, _IMAX),
                      axis=1, keepdims=True)
        out_ref[...] = lax.transpose(idx, (1, 0))


def _tc_only_block(logits_ref, out_ref, accv_ref, accc_ref, *, n, w):
    i = pl.program_id(0)
    nb = pl.num_programs(0)
    b = logits_ref.shape[0]
    _tc_main_block(logits_ref, accv_ref, accc_ref, n=n, c_lo=0, w=w)

    @pl.when(i == nb - 1)
    def _():
        acc_v = accv_ref[...]
        m = jnp.max(acc_v, axis=1, keepdims=True)
        idx = jnp.min(jnp.where(acc_v == m, accc_ref[...], _IMAX),
                      axis=1, keepdims=True)
        out_ref[...] = idx


def _kernel_tc_only(logits):
    b, n = logits.shape
    w = min(8192, -(-n // _CW) * _CW)
    nb = -(-n // w)
    out = pl.pallas_call(
        functools.partial(_tc_only_block, n=n, w=w),
        grid=(nb,),
        in_specs=[pl.BlockSpec((b, w), lambda i: (0, i))],
        out_specs=pl.BlockSpec((b, 1), lambda i: (0, 0)),
        out_shape=jax.ShapeDtypeStruct((b, 1), jnp.int32),
        scratch_shapes=[
            pltpu.VMEM((b, _CW), jnp.float32),
            pltpu.VMEM((b, _CW), jnp.int32),
        ],
    )(logits)
    return out[:, 0]


@jax.jit
def kernel(logits):
    b, n = logits.shape
    if (b, n) != (64, 100000):
        return _kernel_tc_only(logits)

    n_sc = 24576
    w_a = 2048
    nb_a = -(-(n - n_sc) // w_a)
    w_b = 8192
    nb_b = n_sc // w_b

    fb = _sc_noise(b, n, n_sc)

    accv, accc = pl.pallas_call(
        functools.partial(_tc_main_block, n=n, c_lo=n_sc, w=w_a),
        grid=(nb_a,),
        in_specs=[pl.BlockSpec((b, w_a), lambda i: (0, i + n_sc // w_a))],
        out_specs=[pl.BlockSpec((b, _CW), lambda i: (0, 0)),
                   pl.BlockSpec((b, _CW), lambda i: (0, 0))],
        out_shape=[jax.ShapeDtypeStruct((b, _CW), jnp.float32),
                   jax.ShapeDtypeStruct((b, _CW), jnp.int32)],
    )(logits)

    out = pl.pallas_call(
        functools.partial(_tc_tail_block, w=w_b),
        grid=(nb_b,),
        in_specs=[
            pl.BlockSpec((b, w_b), lambda i: (0, i)),
            pl.BlockSpec((w_b // _CW, b, _CW), lambda i: (i, 0, 0)),
            pl.BlockSpec((b, _CW), lambda i: (0, 0)),
            pl.BlockSpec((b, _CW), lambda i: (0, 0)),
        ],
        out_specs=pl.BlockSpec((1, b), lambda i: (0, 0)),
        out_shape=jax.ShapeDtypeStruct((1, b), jnp.int32),
        scratch_shapes=[
            pltpu.VMEM((b, _CW), jnp.float32),
            pltpu.VMEM((b, _CW), jnp.int32),
        ],
    )(logits, fb, accv, accc)
    return out[0]
